# Initial kernel scaffold; baseline (speedup 1.0000x reference)
import jax
import jax.numpy as jnp
from jax.experimental import pallas as pl


def kernel(A, x, W1, b1, W2, b2, sigmoid_param):
    raise NotImplementedError("write your pallas kernel here")


# speedup vs baseline: 6109.7654x; 6109.7654x over previous
import jax
import jax.numpy as jnp
from jax.experimental import pallas as pl

_DN_T = (((0,), (0,)), ((), ()))


def _gcn_body(A_ref, x_ref, W1_ref, b1_ref, W2_ref, b2_ref, sp_ref, out_ref):
    A = A_ref[...]
    n = A.shape[0]
    ones = jnp.ones((n, 1), dtype=jnp.float32)
    deg = jax.lax.dot_general(A, ones, _DN_T,
                              preferred_element_type=jnp.float32) + 1.0
    dis = jax.lax.rsqrt(deg)

    x = x_ref[...]
    h = jnp.dot(x, W1_ref[...], preferred_element_type=jnp.float32)
    u = dis * h
    t = jax.lax.dot_general(A, u, _DN_T, preferred_element_type=jnp.float32)
    g1 = jnp.maximum(dis * (t + u) + b1_ref[...], 0.0)

    h2 = jnp.dot(g1, W2_ref[...], preferred_element_type=jnp.float32)
    u2 = dis * h2
    t2 = jax.lax.dot_general(A, u2, _DN_T, preferred_element_type=jnp.float32)
    g2 = dis * (t2 + u2) + b2_ref[...] + x

    out_ref[...] = jax.nn.sigmoid(sp_ref[0, 0] * g2)


def kernel(A, x, W1, b1, W2, b2, sigmoid_param):
    n, din = x.shape
    out = pl.pallas_call(
        _gcn_body,
        out_shape=jax.ShapeDtypeStruct((n, din), jnp.float32),
    )(A, x, W1, b1.reshape(1, -1), W2, b2.reshape(1, -1),
      sigmoid_param.reshape(1, 1).astype(jnp.float32))
    return out.astype(jnp.float64)
